# baseline (speedup 1.0000x reference)
import jax
import jax.numpy as jnp
from jax.experimental import pallas as pl

N_NODES = 10000
IN_FEAT = 128
OUT_FEAT = 128
NUM_BASES = 8
SUBMAT_IN = 16
SUBMAT_OUT = 16


def _copy_body(src_ref, out_ref):
    out_ref[...] = src_ref[...]


def kernel(x, edge_index, edge_type, norm, weight):
    src = edge_index[0]
    dst = edge_index[1]
    w = jnp.take(weight, edge_type, axis=0)
    w = w.reshape(-1, SUBMAT_IN, SUBMAT_OUT)
    node = jnp.take(x, src, axis=0).reshape(-1, 1, SUBMAT_IN)
    msg = jnp.matmul(node, w).reshape(-1, OUT_FEAT)
    h = jax.ops.segment_sum(msg, dst, num_segments=N_NODES)
    h = h * norm
    return pl.pallas_call(
        _copy_body,
        out_shape=jax.ShapeDtypeStruct(h.shape, h.dtype),
    )(h)


# speedup vs baseline: 6.0169x; 6.0169x over previous
import jax
import jax.numpy as jnp
from jax import lax
from jax.experimental import pallas as pl
from jax.experimental.pallas import tpu as pltpu
from jax.experimental.pallas import tpu_sc as plsc

N_NODES = 10000
N_EDGES = 320000
FEAT = 128
NUM_RELS = 200
NUM_BASES = 8
SUB = 16

N_PAD = 10240
ROWS = 320
SCAN_CHUNK = 3200
N_SCAN = N_EDGES // SCAN_CHUNK
M_CAP = 12288
M_ROWS = M_CAP // 128

_BCAST_DNUMS = lax.GatherDimensionNumbers(
    offset_dims=(), collapsed_slice_dims=(0,), start_index_map=(0,))


def _bcast(vec, idx):
    iv = jnp.full((SUB, 1), idx, jnp.int32)
    return lax.gather(vec, iv, _BCAST_DNUMS, (1,),
                      mode=lax.GatherScatterMode.PROMISE_IN_BOUNDS)


def _body(x_hbm, src_hbm, dst_hbm, et_hbm, norm_hbm, w_hbm, out_hbm,
          h, mids, gtyp, gmids, gsrc, gdst, dstbuf, wbuf, xg, nbuf,
          hist, off, cur, sem):
    c = lax.axis_index("c")
    s = lax.axis_index("s")
    wid = c * 16 + s
    lo = wid * ROWS
    hi = lo + ROWS
    zero16f = jnp.zeros((SUB,), jnp.float32)
    zero16i = jnp.zeros((SUB,), jnp.int32)
    iota16 = lax.iota(jnp.int32, SUB)
    lane0 = iota16 == 0

    def _zero_h(r, _):
        for j in range(FEAT // SUB):
            h[r, pl.ds(j * SUB, SUB)] = zero16f
        return 0
    lax.fori_loop(0, ROWS, _zero_h, 0)

    def _zero_2d(ref):
        def body(r, _):
            for j in range(128 // SUB):
                ref[r, pl.ds(j * SUB, SUB)] = zero16i
            return 0
        lax.fori_loop(0, M_ROWS, body, 0)
    _zero_2d(mids)
    _zero_2d(gmids)
    _zero_2d(gsrc)
    def _hz(t, _):
        hist[t] = 0
        return 0
    lax.fori_loop(0, 256, _hz, 0)

    def _scan_chunk(k, m):
        pltpu.sync_copy(dst_hbm.at[pl.ds(k * SCAN_CHUNK, SCAN_CHUNK)], dstbuf)
        def vec_body(v, m):
            dv = dstbuf[pl.ds(v * SUB, SUB)]
            inr = (dv >= lo) & (dv < hi)
            ii = inr.astype(jnp.int32)
            csum = plsc.cumsum(ii)
            pos = m + csum - 1
            eids = k * SCAN_CHUNK + v * SUB + iota16
            plsc.store_scatter(mids, [pos // 128, pos % 128], eids, mask=inr)
            return jnp.minimum(m + jnp.sum(ii), M_CAP - SUB)
        return lax.fori_loop(0, SCAN_CHUNK // SUB, vec_body, m)
    m = lax.fori_loop(0, N_SCAN, _scan_chunk, jnp.int32(0))

    def _gather_all(table_hbm, idx2d, dst2d):
        for wave in range(0, M_ROWS, 8):
            cps = [pltpu.async_copy(table_hbm.at[idx2d.at[g]], dst2d.at[g], sem)
                   for g in range(wave, min(wave + 8, M_ROWS))]
            for cp in cps:
                cp.wait()
    _gather_all(et_hbm, mids, gtyp)

    ngrp = (m + SUB - 1) // SUB

    def _hist_grp(g, _):
        tv = gtyp[g // 8, pl.ds((g % 8) * SUB, SUB)]
        for l in range(SUB):
            @pl.when(g * SUB + l < m)
            def _():
                t = tv[l]
                hist[t] = hist[t] + 1
        return 0
    lax.fori_loop(0, ngrp, _hist_grp, 0)

    def _pfx(t, run):
        cnt = hist[t]
        off[t] = run
        cur[t] = run
        return run + cnt
    total = lax.fori_loop(0, NUM_RELS, _pfx, jnp.int32(0))
    off[NUM_RELS] = total

    def _group_grp(g, _):
        tv = gtyp[g // 8, pl.ds((g % 8) * SUB, SUB)]
        mv = mids[g // 8, pl.ds((g % 8) * SUB, SUB)]
        for l in range(SUB):
            @pl.when(g * SUB + l < m)
            def _():
                t = tv[l]
                p = cur[t]
                cur[t] = p + 1
                plsc.store_scatter(gmids,
                                   [jnp.full((SUB,), p // 128, jnp.int32),
                                    jnp.full((SUB,), p % 128, jnp.int32)],
                                   jnp.full((SUB,), mv[l], jnp.int32),
                                   mask=lane0)
        return 0
    lax.fori_loop(0, ngrp, _group_grp, 0)

    _gather_all(src_hbm, gmids, gsrc)
    _gather_all(dst_hbm, gmids, gdst)

    def _type_body(t, _):
        pltpu.sync_copy(w_hbm.at[t], wbuf)
        s0 = off[t]
        s1 = off[t + 1]
        def _chunk_body(g, _):
            grow = g // 8
            gcol = (g % 8) * SUB
            pltpu.sync_copy(x_hbm.at[gsrc.at[grow, pl.ds(gcol, SUB)]], xg)
            dv = gdst[grow, pl.ds(gcol, SUB)]
            def _edge_body(e16, _):
                ge = g * SUB + e16
                @pl.when((ge >= s0) & (ge < s1))
                def _():
                    drow = _bcast(dv, e16) - lo
                    for b in range(NUM_BASES):
                        xb = xg[e16, pl.ds(b * SUB, SUB)]
                        acc = zero16f
                        for i in range(SUB):
                            acc = acc + _bcast(xb, i) * wbuf[pl.ds(b * 256 + i * SUB, SUB)]
                        plsc.addupdate_scatter(h, [drow, b * SUB + iota16], acc)
                return 0
            lax.fori_loop(0, SUB, _edge_body, 0)
            return 0
        lax.fori_loop(s0 // SUB, (s1 + SUB - 1) // SUB, _chunk_body, 0)
        return 0
    lax.fori_loop(0, NUM_RELS, _type_body, 0)

    pltpu.sync_copy(norm_hbm.at[pl.ds(lo, ROWS)], nbuf)
    def _norm_grp(g, _):
        nv = nbuf[pl.ds(g * SUB, SUB)]
        for l in range(SUB):
            nvs = nv[l]
            r = g * SUB + l
            for j in range(FEAT // SUB):
                h[r, pl.ds(j * SUB, SUB)] = h[r, pl.ds(j * SUB, SUB)] * nvs
        return 0
    lax.fori_loop(0, ROWS // SUB, _norm_grp, 0)
    pltpu.sync_copy(h, out_hbm.at[pl.ds(lo, ROWS)])


@jax.jit
def _rgcn_sc(x, src, dst, et, norm_pad, weight):
    mesh = plsc.VectorSubcoreMesh(core_axis_name="c", subcore_axis_name="s")
    f = pl.kernel(
        _body,
        out_type=jax.ShapeDtypeStruct((N_PAD, FEAT), jnp.float32),
        mesh=mesh,
        compiler_params=pltpu.CompilerParams(needs_layout_passes=False),
        scratch_types=[
            pltpu.VMEM((ROWS, FEAT), jnp.float32),
            pltpu.VMEM((M_ROWS, 128), jnp.int32),
            pltpu.VMEM((M_ROWS, 128), jnp.int32),
            pltpu.VMEM((M_ROWS, 128), jnp.int32),
            pltpu.VMEM((M_ROWS, 128), jnp.int32),
            pltpu.VMEM((M_ROWS, 128), jnp.int32),
            pltpu.VMEM((SCAN_CHUNK,), jnp.int32),
            pltpu.VMEM((NUM_BASES * SUB * SUB,), jnp.float32),
            pltpu.VMEM((SUB, FEAT), jnp.float32),
            pltpu.VMEM((ROWS,), jnp.float32),
            pltpu.SMEM((256,), jnp.int32),
            pltpu.SMEM((256,), jnp.int32),
            pltpu.SMEM((256,), jnp.int32),
            pltpu.SemaphoreType.DMA,
        ],
    )
    return f(x, src, dst, et, norm_pad, weight)


def kernel(x, edge_index, edge_type, norm, weight):
    src = edge_index[0].astype(jnp.int32)
    dst = edge_index[1].astype(jnp.int32)
    et = edge_type.astype(jnp.int32)
    norm_pad = jnp.pad(norm[:, 0].astype(jnp.float32), (0, N_PAD - N_NODES))
    out = _rgcn_sc(x.astype(jnp.float32), src, dst, et, norm_pad,
                   weight.astype(jnp.float32))
    return out[:N_NODES]
